# baseline (speedup 1.0000x reference)
import functools

import jax
import jax.numpy as jnp
from jax import lax
from jax.experimental import pallas as pl
from jax.experimental.pallas import tpu as pltpu
from jax.experimental.pallas import tpu_sc as plsc

R = 128
N = 32768
NW = 32
ROWS_PW = R // NW
L = 16
CH = N // L
DB = 11
NB = 1 << DB
NB3 = 1 << (32 - 2 * DB)
MIN32 = -(1 << 31)
UN = 8


def _to_mono(v):
    return v ^ ((v >> 31) | MIN32)


def _from_mono(u):
    return u ^ jnp.where(u < 0, MIN32, -1)


def _digit(u, shift, mask_bits):
    uu = plsc.bitcast(u, jnp.uint32)
    uu = uu >> shift if shift else uu
    if mask_bits:
        uu = uu & mask_bits
    return plsc.bitcast(uu, jnp.int32)


def _zero(h, nb):
    def body(i, c):
        h[pl.ds(i * L, L)] = jnp.zeros((L,), jnp.int32)
        return c

    lax.fori_loop(0, nb // L, body, 0, unroll=UN)


def _exscan(h, nb):
    def body(i, carry):
        v = h[pl.ds(i * L, L)]
        inc = plsc.cumsum(v)
        h[pl.ds(i * L, L)] = inc - v + carry
        return carry + jnp.sum(v)

    lax.fori_loop(0, nb // L, body, jnp.full((), -1, jnp.int32), unroll=4)


def _ones():
    return jnp.full((L,), 1, jnp.int32)


def _pipe(n, prefetch, commit):
    def body(i, st):
        nxt = prefetch(i + 1)
        commit(st)
        return nxt

    last = lax.fori_loop(0, n - 1, body, prefetch(0), unroll=UN)
    commit(last)


def _pipe_g(n, prefetch, gather, commit):
    def body(i, carry):
        st, base = carry
        nxt = prefetch(i + 1)
        commit(st, base)
        return nxt, gather(nxt)

    st0 = prefetch(0)
    last, lbase = lax.fori_loop(0, n - 1, body, (st0, gather(st0)),
                                unroll=UN)
    commit(last, lbase)


_mesh = plsc.VectorSubcoreMesh(core_axis_name="c", subcore_axis_name="s")


@functools.partial(
    pl.kernel,
    mesh=_mesh,
    compiler_params=pltpu.CompilerParams(needs_layout_passes=False),
    out_type=jax.ShapeDtypeStruct((R, N), jnp.int32),
    scratch_types=[
        pltpu.VMEM((N,), jnp.int32),
        pltpu.VMEM((N,), jnp.int32),
        pltpu.VMEM((N,), jnp.int32),
        pltpu.VMEM((NB,), jnp.int32),
        pltpu.VMEM((NB,), jnp.int32),
        pltpu.VMEM((NB3,), jnp.int32),
        pltpu.SemaphoreType.DMA,
        pltpu.SemaphoreType.DMA,
    ],
)
def _sort_rows(x_hbm, out_hbm, b0, b1, b2, h1, h2, h3, in_sem, out_sem):
    wid = lax.axis_index("s") * 2 + lax.axis_index("c")
    bufs = (b0, b1, b2)
    rot = [(0, 1, 2), (2, 0, 1), (1, 2, 0), (0, 1, 2)]
    in_h = None
    out_hs = []
    for r in range(ROWS_PW):
        row = wid * ROWS_PW + r
        buf_a = bufs[rot[r][0]]
        buf_b = bufs[rot[r][1]]
        buf_s = bufs[rot[r][2]]
        if r == 0:
            pltpu.sync_copy(x_hbm.at[row], buf_a)
        else:
            in_h.wait()
        if r + 1 < ROWS_PW:
            if out_hs:
                out_hs.pop(0).wait()
            in_h = pltpu.async_copy(x_hbm.at[row + 1], buf_s, in_sem)

        _zero(h1, NB)

        def pre_h1(i):
            u = _to_mono(buf_a[pl.ds(i * L, L)])
            buf_a[pl.ds(i * L, L)] = u
            return (_digit(u, 0, NB - 1),)

        def com_h1(st):
            plsc.addupdate_scatter(h1, [st[0]], _ones())

        _pipe(CH, pre_h1, com_h1)
        _exscan(h1, NB)
        _zero(h2, NB)

        def pre_p1(i):
            u = buf_a[pl.ds(i * L, L)]
            d1 = _digit(u, 0, NB - 1)
            d2 = _digit(u, DB, NB - 1)
            c1, m1 = plsc.scan_count(d1)
            return u, d1, d2, c1, m1

        def g_p1(st):
            return plsc.load_gather(h1, [st[1]])

        def com_p1(st, base):
            u, d1, d2, c1, m1 = st
            plsc.store_scatter(buf_b, [base + c1], u)
            plsc.addupdate_scatter(h1, [d1], c1, mask=m1)
            plsc.addupdate_scatter(h2, [d2], _ones())

        _pipe_g(CH, pre_p1, g_p1, com_p1)
        _exscan(h2, NB)
        _zero(h3, NB3)

        def pre_p2(i):
            u = buf_b[pl.ds(i * L, L)]
            d2 = _digit(u, DB, NB - 1)
            d3 = _digit(u, 2 * DB, 0)
            c2, m2 = plsc.scan_count(d2)
            return u, d2, d3, c2, m2

        def g_p2(st):
            return plsc.load_gather(h2, [st[1]])

        def com_p2(st, base):
            u, d2, d3, c2, m2 = st
            plsc.store_scatter(buf_a, [base + c2], u)
            plsc.addupdate_scatter(h2, [d2], c2, mask=m2)
            plsc.addupdate_scatter(h3, [d3], _ones())

        _pipe_g(CH, pre_p2, g_p2, com_p2)
        _exscan(h3, NB3)

        def pre_p3(i):
            u = buf_a[pl.ds(i * L, L)]
            d3 = _digit(u, 2 * DB, 0)
            c3, m3 = plsc.scan_count(d3)
            return u, d3, c3, m3

        def g_p3(st):
            return plsc.load_gather(h3, [st[1]])

        def com_p3(st, base):
            u, d3, c3, m3 = st
            plsc.store_scatter(buf_b, [base + c3], _from_mono(u))
            plsc.addupdate_scatter(h3, [d3], c3, mask=m3)

        _pipe_g(CH, pre_p3, g_p3, com_p3)
        out_hs.append(pltpu.async_copy(buf_b, out_hbm.at[row], out_sem))
    for h in out_hs:
        h.wait()


def kernel(x):
    xi = lax.bitcast_convert_type(x, jnp.int32)
    yi = _sort_rows(xi)
    return lax.bitcast_convert_type(yi, jnp.float32)


# speedup vs baseline: 1.1113x; 1.1113x over previous
import functools

import jax
import jax.numpy as jnp
from jax import lax
from jax.experimental import pallas as pl
from jax.experimental.pallas import tpu as pltpu
from jax.experimental.pallas import tpu_sc as plsc

R = 128
N = 32768
NW = 32
ROWS_PW = R // NW
L = 16
CH = N // L
DB = 11
NB = 1 << DB
NB3 = 1 << (32 - 2 * DB)
MIN32 = -(1 << 31)
UN = 8


def _to_mono(v):
    return v ^ ((v >> 31) | MIN32)


def _from_mono(u):
    return u ^ jnp.where(u < 0, MIN32, -1)


def _digit(u, shift, mask_bits):
    uu = plsc.bitcast(u, jnp.uint32)
    uu = uu >> shift if shift else uu
    if mask_bits:
        uu = uu & mask_bits
    return plsc.bitcast(uu, jnp.int32)


def _zero(h, nb):
    def body(i, c):
        h[pl.ds(i * L, L)] = jnp.zeros((L,), jnp.int32)
        return c

    lax.fori_loop(0, nb // L, body, 0, unroll=UN)


def _exscan(h, nb):
    def body(i, carry):
        v = h[pl.ds(i * L, L)]
        inc = plsc.cumsum(v)
        h[pl.ds(i * L, L)] = inc - v + carry
        return carry + jnp.sum(v)

    lax.fori_loop(0, nb // L, body, jnp.full((), -1, jnp.int32), unroll=4)


def _ones():
    return jnp.full((L,), 1, jnp.int32)


def _pipe(n, prefetch, commit):
    def body(i, st):
        nxt = prefetch(i + 1)
        commit(st)
        return nxt

    last = lax.fori_loop(0, n - 1, body, prefetch(0), unroll=UN)
    commit(last)


def _pipe_g(n, prefetch, gather, commit):
    def body(i, carry):
        st, base = carry
        nxt = prefetch(i + 1)
        commit(st, base)
        return nxt, gather(nxt)

    st0 = prefetch(0)
    last, lbase = lax.fori_loop(0, n - 1, body, (st0, gather(st0)),
                                unroll=UN)
    commit(last, lbase)


_mesh = plsc.VectorSubcoreMesh(core_axis_name="c", subcore_axis_name="s")


@functools.partial(
    pl.kernel,
    mesh=_mesh,
    compiler_params=pltpu.CompilerParams(needs_layout_passes=False),
    out_type=jax.ShapeDtypeStruct((R, N), jnp.int32),
    scratch_types=[
        pltpu.VMEM((N,), jnp.int32),
        pltpu.VMEM((N,), jnp.int32),
        pltpu.VMEM((N,), jnp.int32),
        pltpu.VMEM((NB,), jnp.int32),
        pltpu.VMEM((NB,), jnp.int32),
        pltpu.VMEM((NB3,), jnp.int32),
        pltpu.SemaphoreType.DMA,
        pltpu.SemaphoreType.DMA,
    ],
)
def _sort_rows(x_hbm, out_hbm, b0, b1, b2, h1, h2, h3, in_sem, out_sem):
    wid = lax.axis_index("s") * 2 + lax.axis_index("c")
    bufs = (b0, b1, b2)
    rot = [(0, 1, 2), (2, 0, 1), (1, 2, 0), (0, 1, 2)]
    in_h = None
    out_hs = []
    for r in range(ROWS_PW):
        row = wid * ROWS_PW + r
        buf_a = bufs[rot[r][0]]
        buf_b = bufs[rot[r][1]]
        buf_s = bufs[rot[r][2]]
        if r == 0:
            pltpu.sync_copy(x_hbm.at[row], buf_a)
        elif in_h is not None:
            in_h.wait()
        if r + 1 < ROWS_PW:
            if out_hs:
                out_hs.pop(0).wait()
            in_h = pltpu.async_copy(x_hbm.at[row + 1], buf_s, in_sem)

        if r == 0:
            _zero(h1, NB)

            def pre_h1(i):
                u = _to_mono(buf_a[pl.ds(i * L, L)])
                buf_a[pl.ds(i * L, L)] = u
                return (_digit(u, 0, NB - 1),)

            def com_h1(st):
                plsc.addupdate_scatter(h1, [st[0]], _ones())

            _pipe(CH, pre_h1, com_h1)
        _exscan(h1, NB)
        _zero(h2, NB)

        def pre_p1(i):
            u = buf_a[pl.ds(i * L, L)]
            d1 = _digit(u, 0, NB - 1)
            d2 = _digit(u, DB, NB - 1)
            c1, m1 = plsc.scan_count(d1)
            return u, d1, d2, c1, m1

        def g_p1(st):
            return plsc.load_gather(h1, [st[1]])

        def com_p1(st, base):
            u, d1, d2, c1, m1 = st
            plsc.store_scatter(buf_b, [base + c1], u)
            plsc.addupdate_scatter(h1, [d1], c1, mask=m1)
            plsc.addupdate_scatter(h2, [d2], _ones())

        _pipe_g(CH, pre_p1, g_p1, com_p1)
        _exscan(h2, NB)
        _zero(h3, NB3)

        def pre_p2(i):
            u = buf_b[pl.ds(i * L, L)]
            d2 = _digit(u, DB, NB - 1)
            d3 = _digit(u, 2 * DB, 0)
            c2, m2 = plsc.scan_count(d2)
            return u, d2, d3, c2, m2

        def g_p2(st):
            return plsc.load_gather(h2, [st[1]])

        def com_p2(st, base):
            u, d2, d3, c2, m2 = st
            plsc.store_scatter(buf_a, [base + c2], u)
            plsc.addupdate_scatter(h2, [d2], c2, mask=m2)
            plsc.addupdate_scatter(h3, [d3], _ones())

        _pipe_g(CH, pre_p2, g_p2, com_p2)
        _exscan(h3, NB3)
        fuse_next = r + 1 < ROWS_PW
        if fuse_next:
            in_h.wait()
            in_h = None
            _zero(h1, NB)

        def pre_p3(i):
            u = buf_a[pl.ds(i * L, L)]
            d3 = _digit(u, 2 * DB, 0)
            c3, m3 = plsc.scan_count(d3)
            if fuse_next:
                un = _to_mono(buf_s[pl.ds(i * L, L)])
                buf_s[pl.ds(i * L, L)] = un
                dn = _digit(un, 0, NB - 1)
            else:
                dn = None
            return u, d3, c3, m3, dn

        def g_p3(st):
            return plsc.load_gather(h3, [st[1]])

        def com_p3(st, base):
            u, d3, c3, m3, dn = st
            plsc.store_scatter(buf_b, [base + c3], _from_mono(u))
            plsc.addupdate_scatter(h3, [d3], c3, mask=m3)
            if fuse_next:
                plsc.addupdate_scatter(h1, [dn], _ones())

        _pipe_g(CH, pre_p3, g_p3, com_p3)
        out_hs.append(pltpu.async_copy(buf_b, out_hbm.at[row], out_sem))
    for h in out_hs:
        h.wait()


def kernel(x):
    xi = lax.bitcast_convert_type(x, jnp.int32)
    yi = _sort_rows(xi)
    return lax.bitcast_convert_type(yi, jnp.float32)
